# Initial kernel scaffold; baseline (speedup 1.0000x reference)
import jax
import jax.numpy as jnp
from jax.experimental import pallas as pl


def kernel(hidden, attention_mask, tok_w, tok_b, cls_w, cls_b, ln_tok_g, ln_tok_b, ln_cls_g, ln_cls_b):
    raise NotImplementedError("write your pallas kernel here")


# speedup vs baseline: 4.1834x; 4.1834x over previous
import jax
import jax.numpy as jnp
from jax.experimental import pallas as pl
from jax.experimental.pallas import tpu as pltpu

EPS = 1e-5
WINDOW = 5


def _shift(x, d):
    if d == 0:
        return x
    cols = x.shape[1]
    if d > 0:
        return jnp.concatenate(
            [x[d:], jnp.zeros((d, cols), x.dtype)], axis=0)
    return jnp.concatenate(
        [jnp.zeros((-d, cols), x.dtype), x[:d]], axis=0)


def _body(h_ref, m_ref, tokw_ref, tokb_ref, clsw_ref, clsb_ref,
          lntg_ref, lntb_ref, lncg_ref, lncb_ref, cls_ref, reps_ref):
    h = h_ref[0]
    S = h.shape[0]
    TD = tokw_ref.shape[1]
    W = WINDOW

    c = jnp.dot(h[0:1, :], clsw_ref[...],
                preferred_element_type=jnp.float32) + clsb_ref[...]
    cm = jnp.mean(c, axis=-1, keepdims=True)
    cv = jnp.mean((c - cm) ** 2, axis=-1, keepdims=True)
    cls_ref[0] = (c - cm) * jax.lax.rsqrt(cv + EPS) * lncg_ref[...] + lncb_ref[...]

    t = jnp.dot(h, tokw_ref[...],
                preferred_element_type=jnp.float32) + tokb_ref[...]
    tm = jnp.mean(t, axis=-1, keepdims=True)
    tv = jnp.mean((t - tm) ** 2, axis=-1, keepdims=True)
    t = (t - tm) * jax.lax.rsqrt(tv + EPS) * lntg_ref[...] + lntb_ref[...]
    r = jnp.maximum(t, 0.0)

    mv = m_ref[0]
    lane = jax.lax.broadcasted_iota(jnp.int32, (1, S), 1)
    L = jnp.sum(jnp.where((lane >= 1) & (lane < S - 1), mv, 0))

    q = jax.lax.broadcasted_iota(jnp.int32, (S, 1), 0)
    rm = jnp.where(q < L,
                   jnp.concatenate([r[1:], jnp.zeros((1, TD), r.dtype)], axis=0),
                   0.0)

    rp = jnp.concatenate([jnp.zeros((8, TD), rm.dtype), rm], axis=0)
    t2 = rp + _shift(rp, 1)
    t4 = t2 + _shift(t2, 2)
    t8 = t4 + _shift(t4, 4)
    ws = (_shift(t8, -W) + _shift(t2, W - 2))[8:]

    cnt = jnp.maximum(jnp.minimum(q + W, L) - jnp.maximum(q - W, 0), 1)
    mean = ws / cnt.astype(jnp.float32)
    mean = jnp.where(q < L, mean, 0.0)

    n = jnp.sqrt(jnp.sum(mean * mean, axis=-1, keepdims=True))
    out = jnp.where(n > 0, mean / jnp.where(n > 0, n, 1.0), 0.0)
    reps_ref[0] = out[:S - 2]


def kernel(hidden, attention_mask, tok_w, tok_b, cls_w, cls_b,
           ln_tok_g, ln_tok_b, ln_cls_g, ln_cls_b):
    B, S, H = hidden.shape
    TD = tok_w.shape[1]
    CD = cls_w.shape[1]

    mask3 = attention_mask.reshape(B, 1, S)
    full = lambda shape: pl.BlockSpec(shape, lambda b: (0,) * len(shape))

    cls3, reps = pl.pallas_call(
        _body,
        grid=(B,),
        in_specs=[
            pl.BlockSpec((1, S, H), lambda b: (b, 0, 0)),
            pl.BlockSpec((1, 1, S), lambda b: (b, 0, 0)),
            full((H, TD)),
            full((1, TD)),
            full((H, CD)),
            full((1, CD)),
            full((1, TD)),
            full((1, TD)),
            full((1, CD)),
            full((1, CD)),
        ],
        out_specs=[
            pl.BlockSpec((1, 1, CD), lambda b: (b, 0, 0)),
            pl.BlockSpec((1, S - 2, TD), lambda b: (b, 0, 0)),
        ],
        out_shape=[
            jax.ShapeDtypeStruct((B, 1, CD), jnp.float32),
            jax.ShapeDtypeStruct((B, S - 2, TD), jnp.float32),
        ],
        compiler_params=pltpu.CompilerParams(
            dimension_semantics=("parallel",),
        ),
        name="coil_core_fused",
    )(hidden, mask3, tok_w, tok_b.reshape(1, TD), cls_w, cls_b.reshape(1, CD),
      ln_tok_g.reshape(1, TD), ln_tok_b.reshape(1, TD),
      ln_cls_g.reshape(1, CD), ln_cls_b.reshape(1, CD))

    return (cls3.reshape(B, CD), reps)


# speedup vs baseline: 4.3244x; 1.0337x over previous
import jax
import jax.numpy as jnp
from jax.experimental import pallas as pl
from jax.experimental.pallas import tpu as pltpu

EPS = 1e-5
WINDOW = 5


def _shift(x, d):
    if d == 0:
        return x
    cols = x.shape[1]
    if d > 0:
        return jnp.concatenate(
            [x[d:], jnp.zeros((d, cols), x.dtype)], axis=0)
    return jnp.concatenate(
        [jnp.zeros((-d, cols), x.dtype), x[:d]], axis=0)


def _body(h_ref, m_ref, tokw_ref, tokb_ref, clsw_ref, clsb_ref,
          lntg_ref, lntb_ref, lncg_ref, lncb_ref, cls_ref, reps_ref):
    h = h_ref[0]
    S = h.shape[0]
    TD = tokw_ref.shape[1]
    W = WINDOW

    c = jnp.dot(h[0:1, :].astype(jnp.bfloat16), clsw_ref[...],
                preferred_element_type=jnp.float32) + clsb_ref[...]
    cm = jnp.mean(c, axis=-1, keepdims=True)
    cv = jnp.mean((c - cm) ** 2, axis=-1, keepdims=True)
    cls_ref[0] = (c - cm) * jax.lax.rsqrt(cv + EPS) * lncg_ref[...] + lncb_ref[...]

    t = jnp.dot(h.astype(jnp.bfloat16), tokw_ref[...],
                preferred_element_type=jnp.float32) + tokb_ref[...]
    tm = jnp.mean(t, axis=-1, keepdims=True)
    tv = jnp.mean((t - tm) ** 2, axis=-1, keepdims=True)
    t = (t - tm) * jax.lax.rsqrt(tv + EPS) * lntg_ref[...] + lntb_ref[...]
    r = jnp.maximum(t, 0.0)

    mv = m_ref[0]
    lane = jax.lax.broadcasted_iota(jnp.int32, (1, S), 1)
    L = jnp.sum(jnp.where((lane >= 1) & (lane < S - 1), mv, 0))

    q = jax.lax.broadcasted_iota(jnp.int32, (S, 1), 0)
    rm = jnp.where(q < L,
                   jnp.concatenate([r[1:], jnp.zeros((1, TD), r.dtype)], axis=0),
                   0.0)

    rp = jnp.concatenate([jnp.zeros((8, TD), rm.dtype), rm], axis=0)
    t2 = rp + _shift(rp, 1)
    t4 = t2 + _shift(t2, 2)
    t8 = t4 + _shift(t4, 4)
    ws = (_shift(t8, -W) + _shift(t2, W - 2))[8:]

    n2 = jnp.sum(ws * ws, axis=-1, keepdims=True)
    scale = jnp.where((q < L) & (n2 > 0), jax.lax.rsqrt(n2), 0.0)
    reps_ref[0] = (ws * scale)[:S - 2]


def kernel(hidden, attention_mask, tok_w, tok_b, cls_w, cls_b,
           ln_tok_g, ln_tok_b, ln_cls_g, ln_cls_b):
    B, S, H = hidden.shape
    TD = tok_w.shape[1]
    CD = cls_w.shape[1]

    mask3 = attention_mask.reshape(B, 1, S)
    full = lambda shape: pl.BlockSpec(shape, lambda b: (0,) * len(shape))

    cls3, reps = pl.pallas_call(
        _body,
        grid=(B,),
        in_specs=[
            pl.BlockSpec((1, S, H), lambda b: (b, 0, 0)),
            pl.BlockSpec((1, 1, S), lambda b: (b, 0, 0)),
            full((H, TD)),
            full((1, TD)),
            full((H, CD)),
            full((1, CD)),
            full((1, TD)),
            full((1, TD)),
            full((1, CD)),
            full((1, CD)),
        ],
        out_specs=[
            pl.BlockSpec((1, 1, CD), lambda b: (b, 0, 0)),
            pl.BlockSpec((1, S - 2, TD), lambda b: (b, 0, 0)),
        ],
        out_shape=[
            jax.ShapeDtypeStruct((B, 1, CD), jnp.float32),
            jax.ShapeDtypeStruct((B, S - 2, TD), jnp.float32),
        ],
        compiler_params=pltpu.CompilerParams(
            dimension_semantics=("parallel",),
        ),
        name="coil_core_fused",
    )(hidden, mask3, tok_w.astype(jnp.bfloat16), tok_b.reshape(1, TD),
      cls_w.astype(jnp.bfloat16), cls_b.reshape(1, CD),
      ln_tok_g.reshape(1, TD), ln_tok_b.reshape(1, TD),
      ln_cls_g.reshape(1, CD), ln_cls_b.reshape(1, CD))

    return (cls3.reshape(B, CD), reps)
